# baseline (speedup 1.0000x reference)
import functools

import jax
import jax.numpy as jnp
from jax.experimental import pallas as pl
from jax.experimental.pallas import tpu as pltpu


def _loss_kernel(total_cells, nsteps, cx_ref, cy_ref, bx_ref, by_ref,
                 out_ref, acc_ref):
    step = pl.program_id(0)

    @pl.when(step == 0)
    def _init():
        acc_ref[0] = 0.0
        acc_ref[1] = 0.0
        acc_ref[2] = 0.0
        acc_ref[3] = 0.0

    cx = cx_ref[...]
    cy = cy_ref[...]
    mask = (cy > 0.5).astype(jnp.float32)
    face_num = jnp.sum(mask)

    sq = jnp.zeros_like(cx)
    for c in range(4):
        d = bx_ref[c] - by_ref[c]
        sq = sq + d * d
    mse_sum = jnp.sum(mask * sq)

    logp = jnp.maximum(jnp.log(cx), -100.0)
    log1mp = jnp.maximum(jnp.log(1.0 - cx), -100.0)
    bce_pos_sum = jnp.sum(mask * -(cy * logp + (1.0 - cy) * log1mp))
    bce_bg_sum = jnp.sum((mask - 1.0) * log1mp)

    acc_ref[0] += face_num
    acc_ref[1] += mse_sum
    acc_ref[2] += bce_pos_sum
    acc_ref[3] += bce_bg_sum

    @pl.when(step == nsteps - 1)
    def _finalize():
        f = acc_ref[0]
        bg_num = total_cells - f
        inv_f = 1.0 / f
        loss = (1.0 + inv_f) * ((0.25 * acc_ref[1] + acc_ref[2]) * inv_f)
        out_ref[0, 0] = loss + acc_ref[3] / bg_num


@jax.jit
def kernel(x, y):
    B, N, C = x.shape
    npad = (-N) % 128
    NP = N + npad

    cx = jnp.pad(x[:, :, 0], ((0, 0), (0, npad)))
    cy = jnp.pad(y[:, :, 0], ((0, 0), (0, npad)))
    bx = jnp.pad(jnp.transpose(x[:, :, 1:5], (2, 0, 1)), ((0, 0), (0, 0), (0, npad)))
    by = jnp.pad(jnp.transpose(y[:, :, 1:5], (2, 0, 1)), ((0, 0), (0, 0), (0, npad)))

    nsteps = 8
    rb = B // nsteps

    out = pl.pallas_call(
        functools.partial(_loss_kernel, float(B * N), nsteps),
        grid=(nsteps,),
        out_shape=jax.ShapeDtypeStruct((1, 1), jnp.float32),
        in_specs=[
            pl.BlockSpec((rb, NP), lambda i: (i, 0)),
            pl.BlockSpec((rb, NP), lambda i: (i, 0)),
            pl.BlockSpec((4, rb, NP), lambda i: (0, i, 0)),
            pl.BlockSpec((4, rb, NP), lambda i: (0, i, 0)),
        ],
        out_specs=pl.BlockSpec(memory_space=pltpu.SMEM),
        scratch_shapes=[pltpu.SMEM((4,), jnp.float32)],
    )(cx, cy, bx, by)
    return out[0, 0]
